# baseline (speedup 1.0000x reference)
import jax
import jax.numpy as jnp
from jax import lax
from jax.experimental import pallas as pl
from jax.experimental.pallas import tpu as pltpu
from jax.experimental.pallas import tpu_sc as plsc

MAX_LEN = 200
EMB_DIM = 32
BATCH = 4096
N_IDX = BATCH * MAX_LEN
N_GRP = MAX_LEN // 4
N_ROWS = N_IDX // 4

_info = plsc.get_sparse_core_info()
NC, NS = _info.num_cores, _info.num_subcores
NW = NC * NS
ROWS_W = N_ROWS // NW
RCHUNK = 256
N_CHUNKS = ROWS_W // RCHUNK


def _gather_body(x_hbm, table_hbm, out_hbm, idx_v, rows_v, sem):
    wid = lax.axis_index("s") * NC + lax.axis_index("c")
    base = wid * ROWS_W

    def chunk_body(i, carry):
        row0 = base + i * RCHUNK
        for r in range(4):
            pltpu.sync_copy(
                x_hbm.at[r, pl.ds(row0, RCHUNK)], idx_v.at[r]
            )
        for r in range(4):
            pltpu.async_copy(table_hbm.at[idx_v.at[r]], rows_v.at[r], sem)
        for r in range(4):
            pltpu.make_async_copy(
                table_hbm.at[idx_v.at[r]], rows_v.at[r], sem
            ).wait()
        for r in range(4):
            pltpu.sync_copy(
                rows_v.at[r],
                out_hbm.at[pl.ds(row0, RCHUNK), pl.ds(r * EMB_DIM, EMB_DIM)],
            )
        return carry

    lax.fori_loop(0, N_CHUNKS, chunk_body, 0)


def _sc_gather(x_q, table):
    mesh = plsc.VectorSubcoreMesh(core_axis_name="c", subcore_axis_name="s")
    kern = pl.kernel(
        _gather_body,
        mesh=mesh,
        out_type=jax.ShapeDtypeStruct((N_ROWS, 128), jnp.float32),
        scratch_types=[
            pltpu.VMEM((4, RCHUNK), jnp.int32),
            pltpu.VMEM((4, RCHUNK, EMB_DIM), jnp.float32),
            pltpu.SemaphoreType.DMA,
        ],
        compiler_params=pltpu.CompilerParams(use_tc_tiling_on_sc=False),
    )
    return kern(x_q, table)


BB = 512
NB = BATCH // BB


def _mlp_body(emb_ref, w1_ref, b1_ref, w2_ref, b2_ref, out_ref, acc_ref):
    k = pl.program_id(1)

    @pl.when(k == 0)
    def _():
        acc_ref[...] = jnp.zeros_like(acc_ref)

    acc_ref[...] += jnp.dot(
        emb_ref[...], w1_ref[...], preferred_element_type=jnp.float32
    )

    @pl.when(k == N_GRP - 1)
    def _():
        h = jnp.maximum(acc_ref[...] + b1_ref[...], 0.0)
        o = jnp.dot(h, w2_ref[...], preferred_element_type=jnp.float32)
        out_ref[...] = jax.nn.sigmoid(o + b2_ref[...])


def _tc_mlp(emb128, w1t, b1, w2t, b2):
    f = pl.pallas_call(
        _mlp_body,
        grid=(NB, N_GRP),
        in_specs=[
            pl.BlockSpec((BB, 128), lambda i, k: (k * NB + i, 0)),
            pl.BlockSpec((128, 32), lambda i, k: (k, 0)),
            pl.BlockSpec((1, 32), lambda i, k: (0, 0)),
            pl.BlockSpec((32, 1), lambda i, k: (0, 0)),
            pl.BlockSpec((1, 1), lambda i, k: (0, 0)),
        ],
        out_specs=pl.BlockSpec((BB, 1), lambda i, k: (i, 0)),
        out_shape=jax.ShapeDtypeStruct((BATCH, 1), jnp.float32),
        scratch_shapes=[pltpu.VMEM((BB, 32), jnp.float32)],
    )
    return f(emb128, w1t, b1, w2t, b2)


@jax.jit
def kernel(x, table, W1, b1, W2, b2):
    x_q = x.reshape(BATCH, N_GRP, 4).transpose(2, 1, 0).reshape(4, N_ROWS)
    emb128 = _sc_gather(x_q, table)
    return _tc_mlp(emb128, W1.T, b1.reshape(1, 32), W2.T, b2.reshape(1, 1))


# speedup vs baseline: 1.0132x; 1.0132x over previous
import jax
import jax.numpy as jnp
from jax import lax
from jax.experimental import pallas as pl
from jax.experimental.pallas import tpu as pltpu
from jax.experimental.pallas import tpu_sc as plsc

MAX_LEN = 200
EMB_DIM = 32
BATCH = 4096
N_IDX = BATCH * MAX_LEN
N_GRP = MAX_LEN // 4
N_ROWS = N_IDX // 4
VOCAB = 1000000
TROWS = VOCAB // 4

_info = plsc.get_sparse_core_info()
NC, NS = _info.num_cores, _info.num_subcores
NW = NC * NS

TCHUNK = 1024
ALIGNED = (VOCAB // 128) * 128 - ((VOCAB // 128) * 128) % TCHUNK
N_TCHUNKS = ALIGNED // TCHUNK
TAIL = (VOCAB // 128) * 128 - ALIGNED
REM = VOCAB - ALIGNED - TAIL
TITER = (N_TCHUNKS + NW - 1) // NW


def _transpose_chunk(tt_hbm, out_hbm, in_v, out_v, n_tok, t0, r0):
    lane = jax.lax.iota(jnp.int32, 16)
    rbase = lane >> 2
    fquarter = (lane & 3) * EMB_DIM

    def group_body(g, carry):
        j0 = g * 16
        row_idx = rbase + (j0 >> 2)
        for d in range(EMB_DIM):
            v = in_v[d, pl.ds(j0, 16)]
            plsc.store_scatter(out_v, [row_idx, fquarter + d], v)
        return carry

    pltpu.sync_copy(tt_hbm.at[:, pl.ds(t0, n_tok)], in_v.at[:, pl.ds(0, n_tok)])
    lax.fori_loop(0, n_tok // 16, group_body, 0)
    pltpu.sync_copy(
        out_v.at[pl.ds(0, n_tok // 4)], out_hbm.at[pl.ds(r0, n_tok // 4)]
    )


def _transpose_body(tt_hbm, tail_hbm, out_hbm, in_v, out_v):
    wid = lax.axis_index("s") * NC + lax.axis_index("c")

    def chunk_body(i, carry):
        c = i * NW + wid

        @pl.when(c < N_TCHUNKS)
        def _():
            _transpose_chunk(
                tt_hbm, out_hbm, in_v, out_v, TCHUNK, c * TCHUNK,
                c * (TCHUNK // 4),
            )

        return carry

    lax.fori_loop(0, TITER, chunk_body, 0)

    @pl.when(wid == 16)
    def _():
        _transpose_chunk(
            tt_hbm, out_hbm, in_v, out_v, TAIL, ALIGNED, ALIGNED // 4
        )

    @pl.when(wid == 17)
    def _():
        pltpu.sync_copy(tail_hbm, out_v.at[pl.ds(0, REM // 4)])
        pltpu.sync_copy(
            out_v.at[pl.ds(0, REM // 4)],
            out_hbm.at[pl.ds((ALIGNED + TAIL) // 4, REM // 4)],
        )


def _sc_transpose(tableT, tail128):
    mesh = plsc.VectorSubcoreMesh(core_axis_name="c", subcore_axis_name="s")
    kern = pl.kernel(
        _transpose_body,
        mesh=mesh,
        out_type=jax.ShapeDtypeStruct((TROWS, 128), jnp.float32),
        scratch_types=[
            pltpu.VMEM((EMB_DIM, TCHUNK), jnp.float32),
            pltpu.VMEM((TCHUNK // 4, 128), jnp.float32),
        ],
        compiler_params=pltpu.CompilerParams(
            use_tc_tiling_on_sc=True, needs_layout_passes=False
        ),
    )
    return kern(tableT, tail128)


PER_W = N_IDX // NW
CHUNK = 1024
N_CHUNKS = PER_W // CHUNK


def _gather_body(x_hbm, table_hbm, out_hbm, idx_v, rows_v, sem):
    wid = lax.axis_index("s") * NC + lax.axis_index("c")
    base = wid * PER_W

    def chunk_body(i, carry):
        off = base + i * CHUNK
        pltpu.sync_copy(x_hbm.at[pl.ds(off, CHUNK)], idx_v)
        pltpu.async_copy(table_hbm.at[idx_v], rows_v, sem).wait()
        pltpu.sync_copy(rows_v, out_hbm.at[pl.ds(off, CHUNK)])
        return carry

    lax.fori_loop(0, N_CHUNKS, chunk_body, 0)


def _sc_gather(x_km, table_lin):
    mesh = plsc.VectorSubcoreMesh(core_axis_name="c", subcore_axis_name="s")
    kern = pl.kernel(
        _gather_body,
        mesh=mesh,
        out_type=jax.ShapeDtypeStruct((N_IDX, EMB_DIM), jnp.float32),
        scratch_types=[
            pltpu.VMEM((CHUNK,), jnp.int32),
            pltpu.VMEM((CHUNK, EMB_DIM), jnp.float32),
            pltpu.SemaphoreType.DMA,
        ],
        compiler_params=pltpu.CompilerParams(use_tc_tiling_on_sc=False),
    )
    return kern(x_km, table_lin)


KPER = 5
NSTEP = N_GRP // KPER


def _mlp_body(emb_ref, w1_ref, b1_ref, w2_ref, b2_ref, out_ref, acc_ref):
    k0 = pl.program_id(0)

    @pl.when(k0 == 0)
    def _():
        acc_ref[...] = jnp.zeros_like(acc_ref)

    acc = acc_ref[...]
    for s in range(KPER):
        acc += jnp.dot(
            emb_ref[pl.ds(s * BATCH, BATCH), :],
            w1_ref[pl.ds(s * 128, 128), :],
            preferred_element_type=jnp.float32,
        )
    acc_ref[...] = acc

    @pl.when(k0 == NSTEP - 1)
    def _():
        h = jnp.maximum(acc + b1_ref[...], 0.0)
        o = jnp.dot(h, w2_ref[...], preferred_element_type=jnp.float32)
        out_ref[...] = jax.nn.sigmoid(o + b2_ref[...])


def _tc_mlp(emb128, w1t, b1, w2t, b2):
    f = pl.pallas_call(
        _mlp_body,
        grid=(NSTEP,),
        in_specs=[
            pl.BlockSpec((KPER * BATCH, 128), lambda k0: (k0, 0)),
            pl.BlockSpec((KPER * 128, 32), lambda k0: (k0, 0)),
            pl.BlockSpec((1, 32), lambda k0: (0, 0)),
            pl.BlockSpec((32, 1), lambda k0: (0, 0)),
            pl.BlockSpec((1, 1), lambda k0: (0, 0)),
        ],
        out_specs=pl.BlockSpec((BATCH, 1), lambda k0: (0, 0)),
        out_shape=jax.ShapeDtypeStruct((BATCH, 1), jnp.float32),
        scratch_shapes=[pltpu.VMEM((BATCH, 32), jnp.float32)],
    )
    return f(emb128, w1t, b1, w2t, b2)


@jax.jit
def kernel(x, table, W1, b1, W2, b2):
    tail128 = table[ALIGNED + TAIL :].reshape(REM // 4, 128)
    tbl128 = _sc_transpose(table.T, tail128)
    table_lin = tbl128.reshape(VOCAB, EMB_DIM)
    x_km = x.reshape(BATCH, N_GRP, 4).transpose(1, 0, 2).reshape(-1)
    emb128 = _sc_gather(x_km, table_lin).reshape(N_ROWS, 128)
    return _tc_mlp(emb128, W1.T, b1.reshape(1, 32), W2.T, b2.reshape(1, 1))


# speedup vs baseline: 1.1318x; 1.1171x over previous
import jax
import jax.numpy as jnp
from jax import lax
from jax.experimental import pallas as pl
from jax.experimental.pallas import tpu as pltpu
from jax.experimental.pallas import tpu_sc as plsc

MAX_LEN = 200
EMB_DIM = 32
BATCH = 4096
N_IDX = BATCH * MAX_LEN
N_GRP = MAX_LEN // 4
N_ROWS = N_IDX // 4
VOCAB = 1000000
TROWS = VOCAB // 4

_info = plsc.get_sparse_core_info()
NC, NS = _info.num_cores, _info.num_subcores
NW = NC * NS

TCHUNK = 1024
ALIGNED = 999424
N_TCHUNKS = ALIGNED // TCHUNK
TAIL = 512
REM = VOCAB - ALIGNED - TAIL
TITER = (N_TCHUNKS + NW - 1) // NW


def _transpose_chunk(tt_hbm, out_hbm, in_v, out_v, n_tok, t0, r0):
    lane = lax.iota(jnp.int32, 16)
    rbase = lane >> 2
    fquarter = (lane & 3) * EMB_DIM

    pltpu.sync_copy(tt_hbm.at[:, pl.ds(t0, n_tok)], in_v.at[:, pl.ds(0, n_tok)])

    @plsc.parallel_loop(0, n_tok // 16, unroll=2)
    def _(g):
        j0 = g * 16
        row_idx = rbase + g * 4
        for d in range(EMB_DIM):
            v = in_v[d, pl.ds(j0, 16)]
            plsc.store_scatter(out_v, [row_idx, fquarter + d], v)

    pltpu.sync_copy(
        out_v.at[pl.ds(0, n_tok // 4)], out_hbm.at[pl.ds(r0, n_tok // 4)]
    )


def _transpose_body(tt_hbm, tail_hbm, out_hbm, in_v, out_v):
    wid = lax.axis_index("s") * NC + lax.axis_index("c")

    def chunk_body(i, carry):
        c = i * NW + wid

        @pl.when(c < N_TCHUNKS)
        def _():
            _transpose_chunk(
                tt_hbm, out_hbm, in_v, out_v, TCHUNK, c * TCHUNK,
                c * (TCHUNK // 4),
            )

        return carry

    lax.fori_loop(0, TITER, chunk_body, 0)

    @pl.when(wid == 16)
    def _():
        _transpose_chunk(
            tt_hbm, out_hbm, in_v, out_v, TAIL, ALIGNED, ALIGNED // 4
        )

    @pl.when(wid == 17)
    def _():
        pltpu.sync_copy(tail_hbm, out_v.at[pl.ds(0, REM // 4)])
        pltpu.sync_copy(
            out_v.at[pl.ds(0, REM // 4)],
            out_hbm.at[pl.ds((ALIGNED + TAIL) // 4, REM // 4)],
        )


def _sc_transpose(tableT, tail128):
    mesh = plsc.VectorSubcoreMesh(core_axis_name="c", subcore_axis_name="s")
    kern = pl.kernel(
        _transpose_body,
        mesh=mesh,
        out_type=jax.ShapeDtypeStruct((TROWS, 128), jnp.float32),
        scratch_types=[
            pltpu.VMEM((EMB_DIM, TCHUNK), jnp.float32),
            pltpu.VMEM((TCHUNK // 4, 128), jnp.float32),
        ],
        compiler_params=pltpu.CompilerParams(
            use_tc_tiling_on_sc=True, needs_layout_passes=False
        ),
    )
    return kern(tableT, tail128)


BBLK = 256
GCHUNKS = N_GRP * (BATCH // BBLK)
G_PER_W = GCHUNKS // NW
NBB = BATCH // BBLK


def _gather_body(x_hbm, table_hbm, out_hbm, idx4_v, idxi_v, rows_v, sem):
    wid = lax.axis_index("s") * NC + lax.axis_index("c")
    lane = lax.iota(jnp.int32, 16)
    lr = lane & 3
    lm = lane >> 2

    def chunk_body(i, carry):
        c = wid * G_PER_W + i
        k = c >> 4
        b0 = (c & (NBB - 1)) * BBLK
        for r in range(4):
            pltpu.sync_copy(
                x_hbm.at[4 * k + r, pl.ds(b0, BBLK)], idx4_v.at[r]
            )

        @plsc.parallel_loop(0, 4 * BBLK // 16, unroll=2)
        def _(g):
            n0 = g * 16
            v = plsc.load_gather(idx4_v, [lr, (n0 >> 2) + lm])
            idxi_v[pl.ds(n0, 16)] = v

        pltpu.async_copy(table_hbm.at[idxi_v], rows_v, sem).wait()
        pltpu.sync_copy(
            rows_v, out_hbm.at[pl.ds(k * (4 * BATCH) + 4 * b0, 4 * BBLK)]
        )
        return carry

    lax.fori_loop(0, G_PER_W, chunk_body, 0)


def _sc_gather(x2d, table_lin):
    mesh = plsc.VectorSubcoreMesh(core_axis_name="c", subcore_axis_name="s")
    kern = pl.kernel(
        _gather_body,
        mesh=mesh,
        out_type=jax.ShapeDtypeStruct((N_IDX, EMB_DIM), jnp.float32),
        scratch_types=[
            pltpu.VMEM((4, BBLK), jnp.int32),
            pltpu.VMEM((4 * BBLK,), jnp.int32),
            pltpu.VMEM((4 * BBLK, EMB_DIM), jnp.float32),
            pltpu.SemaphoreType.DMA,
        ],
        compiler_params=pltpu.CompilerParams(
            use_tc_tiling_on_sc=False, needs_layout_passes=False
        ),
    )
    return kern(x2d, table_lin)


KPER = 5
NSTEP = N_GRP // KPER


def _mlp_body(emb_ref, w1_ref, b1_ref, w2_ref, b2_ref, out_ref, acc_ref):
    k0 = pl.program_id(0)

    @pl.when(k0 == 0)
    def _():
        acc_ref[...] = jnp.zeros_like(acc_ref)

    acc = acc_ref[...]
    for s in range(KPER):
        acc += jnp.dot(
            emb_ref[pl.ds(s * BATCH, BATCH), :],
            w1_ref[pl.ds(s * 128, 128), :],
            preferred_element_type=jnp.float32,
        )
    acc_ref[...] = acc

    @pl.when(k0 == NSTEP - 1)
    def _():
        h = jnp.maximum(acc + b1_ref[...], 0.0)
        o = jnp.dot(h, w2_ref[...], preferred_element_type=jnp.float32)
        out_ref[...] = jax.nn.sigmoid(o + b2_ref[...])


def _tc_mlp(emb128, w1t, b1, w2t, b2):
    f = pl.pallas_call(
        _mlp_body,
        grid=(NSTEP,),
        in_specs=[
            pl.BlockSpec((KPER * BATCH, 128), lambda k0: (k0, 0)),
            pl.BlockSpec((KPER * 128, 32), lambda k0: (k0, 0)),
            pl.BlockSpec((1, 32), lambda k0: (0, 0)),
            pl.BlockSpec((32, 1), lambda k0: (0, 0)),
            pl.BlockSpec((1, 1), lambda k0: (0, 0)),
        ],
        out_specs=pl.BlockSpec((BATCH, 1), lambda k0: (0, 0)),
        out_shape=jax.ShapeDtypeStruct((BATCH, 1), jnp.float32),
        scratch_shapes=[pltpu.VMEM((BATCH, 32), jnp.float32)],
    )
    return f(emb128, w1t, b1, w2t, b2)


@jax.jit
def kernel(x, table, W1, b1, W2, b2):
    tail128 = table[ALIGNED + TAIL :].reshape(REM // 4, 128)
    tbl128 = _sc_transpose(table.T, tail128)
    table_lin = tbl128.reshape(VOCAB, EMB_DIM)
    emb128 = _sc_gather(x.T, table_lin).reshape(N_ROWS, 128)
    return _tc_mlp(emb128, W1.T, b1.reshape(1, 32), W2.T, b2.reshape(1, 1))


# speedup vs baseline: 1.2580x; 1.1115x over previous
import jax
import jax.numpy as jnp
from jax import lax
from jax.experimental import pallas as pl
from jax.experimental.pallas import tpu as pltpu
from jax.experimental.pallas import tpu_sc as plsc

MAX_LEN = 200
EMB_DIM = 32
BATCH = 4096
N_IDX = BATCH * MAX_LEN
N_GRP = MAX_LEN // 4
N_ROWS = N_IDX // 4
VOCAB = 1000000
TROWS = VOCAB // 4

_info = plsc.get_sparse_core_info()
NC, NS = _info.num_cores, _info.num_subcores
NW = NC * NS

TCHUNK = 1024
ALIGNED = 999424
N_TCHUNKS = ALIGNED // TCHUNK
TAIL = 512
REM = VOCAB - ALIGNED - TAIL
TITER = (N_TCHUNKS + NW - 1) // NW


def _transpose_chunk(tt_hbm, out_hbm, in_v, out_v, n_tok, t0, r0):
    lane = lax.iota(jnp.int32, 16)

    pltpu.sync_copy(tt_hbm.at[:, pl.ds(t0, n_tok)], in_v.at[:, pl.ds(0, n_tok)])

    @plsc.parallel_loop(0, n_tok // 4, unroll=2)
    def _(r):
        for q in range(4):
            j = 4 * r + q
            for h in range(2):
                v = plsc.load_gather(in_v, [16 * h + lane, lane * 0 + j])
                out_v[r, pl.ds(q * EMB_DIM + 16 * h, 16)] = v

    pltpu.sync_copy(
        out_v.at[pl.ds(0, n_tok // 4)], out_hbm.at[pl.ds(r0, n_tok // 4)]
    )


def _transpose_body(tt_hbm, tail_hbm, out_hbm, in_v, out_v):
    wid = lax.axis_index("s") * NC + lax.axis_index("c")

    def chunk_body(i, carry):
        c = i * NW + wid

        @pl.when(c < N_TCHUNKS)
        def _():
            _transpose_chunk(
                tt_hbm, out_hbm, in_v, out_v, TCHUNK, c * TCHUNK,
                c * (TCHUNK // 4),
            )

        return carry

    lax.fori_loop(0, TITER, chunk_body, 0)

    @pl.when(wid == 16)
    def _():
        _transpose_chunk(
            tt_hbm, out_hbm, in_v, out_v, TAIL, ALIGNED, ALIGNED // 4
        )

    @pl.when(wid == 17)
    def _():
        pltpu.sync_copy(tail_hbm, out_v.at[pl.ds(0, REM // 4)])
        pltpu.sync_copy(
            out_v.at[pl.ds(0, REM // 4)],
            out_hbm.at[pl.ds((ALIGNED + TAIL) // 4, REM // 4)],
        )


def _sc_transpose(tableT, tail128):
    mesh = plsc.VectorSubcoreMesh(core_axis_name="c", subcore_axis_name="s")
    kern = pl.kernel(
        _transpose_body,
        mesh=mesh,
        out_type=jax.ShapeDtypeStruct((TROWS, 128), jnp.float32),
        scratch_types=[
            pltpu.VMEM((EMB_DIM, TCHUNK + 1), jnp.float32),
            pltpu.VMEM((TCHUNK // 4, 128), jnp.float32),
        ],
        compiler_params=pltpu.CompilerParams(
            use_tc_tiling_on_sc=True, needs_layout_passes=False
        ),
    )
    return kern(tableT, tail128)


BBLK = 256
GCHUNKS = N_GRP * (BATCH // BBLK)
G_PER_W = GCHUNKS // NW
NBB = BATCH // BBLK


def _gather_body(x_hbm, table_hbm, out_hbm, idx4_v, idxi_v, rows_v, sem):
    wid = lax.axis_index("s") * NC + lax.axis_index("c")
    lane = lax.iota(jnp.int32, 16)
    lr = lane & 3
    lm = lane >> 2

    def chunk_body(i, carry):
        c = wid * G_PER_W + i
        k = c >> 4
        b0 = (c & (NBB - 1)) * BBLK
        for r in range(4):
            pltpu.sync_copy(x_hbm.at[4 * k + r, pl.ds(b0, BBLK)], idx4_v.at[r])

        @plsc.parallel_loop(0, BBLK // 16, unroll=2)
        def _(g):
            m0 = g * 16
            for r in range(4):
                v = idx4_v[r, pl.ds(m0, 16)]
                plsc.store_scatter(idxi_v, [4 * (m0 + lane) + r], v)

        pltpu.async_copy(table_hbm.at[idxi_v], rows_v, sem).wait()
        pltpu.sync_copy(
            rows_v, out_hbm.at[pl.ds(k * (4 * BATCH) + 4 * b0, 4 * BBLK)]
        )
        return carry

    lax.fori_loop(0, G_PER_W, chunk_body, 0)


def _sc_gather(x2d, table_lin):
    mesh = plsc.VectorSubcoreMesh(core_axis_name="c", subcore_axis_name="s")
    kern = pl.kernel(
        _gather_body,
        mesh=mesh,
        out_type=jax.ShapeDtypeStruct((N_IDX, EMB_DIM), jnp.float32),
        scratch_types=[
            pltpu.VMEM((4, BBLK), jnp.int32),
            pltpu.VMEM((4 * BBLK,), jnp.int32),
            pltpu.VMEM((4 * BBLK, EMB_DIM), jnp.float32),
            pltpu.SemaphoreType.DMA,
        ],
        compiler_params=pltpu.CompilerParams(
            use_tc_tiling_on_sc=False, needs_layout_passes=False
        ),
    )
    return kern(x2d, table_lin)


KPER = 5
NSTEP = N_GRP // KPER


def _mlp_body(emb_ref, w1_ref, b1_ref, w2_ref, b2_ref, out_ref, acc_ref):
    k0 = pl.program_id(0)

    @pl.when(k0 == 0)
    def _():
        acc_ref[...] = jnp.zeros_like(acc_ref)

    acc = acc_ref[...]
    for s in range(KPER):
        acc += jnp.dot(
            emb_ref[pl.ds(s * BATCH, BATCH), :],
            w1_ref[pl.ds(s * 128, 128), :],
            preferred_element_type=jnp.float32,
        )
    acc_ref[...] = acc

    @pl.when(k0 == NSTEP - 1)
    def _():
        h = jnp.maximum(acc + b1_ref[...], 0.0)
        o = jnp.dot(h, w2_ref[...], preferred_element_type=jnp.float32)
        out_ref[...] = jax.nn.sigmoid(o + b2_ref[...])


def _tc_mlp(emb128, w1t, b1, w2t, b2):
    f = pl.pallas_call(
        _mlp_body,
        grid=(NSTEP,),
        in_specs=[
            pl.BlockSpec((KPER * BATCH, 128), lambda k0: (k0, 0)),
            pl.BlockSpec((KPER * 128, 32), lambda k0: (k0, 0)),
            pl.BlockSpec((1, 32), lambda k0: (0, 0)),
            pl.BlockSpec((32, 1), lambda k0: (0, 0)),
            pl.BlockSpec((1, 1), lambda k0: (0, 0)),
        ],
        out_specs=pl.BlockSpec((BATCH, 1), lambda k0: (0, 0)),
        out_shape=jax.ShapeDtypeStruct((BATCH, 1), jnp.float32),
        scratch_shapes=[pltpu.VMEM((BATCH, 32), jnp.float32)],
    )
    return f(emb128, w1t, b1, w2t, b2)


@jax.jit
def kernel(x, table, W1, b1, W2, b2):
    tail128 = table[ALIGNED + TAIL :].reshape(REM // 4, 128)
    tbl128 = _sc_transpose(table.T, tail128)
    table_lin = tbl128.reshape(VOCAB, EMB_DIM)
    emb128 = _sc_gather(x.T, table_lin).reshape(N_ROWS, 128)
    return _tc_mlp(emb128, W1.T, b1.reshape(1, 32), W2.T, b2.reshape(1, 1))


# speedup vs baseline: 1.5677x; 1.2461x over previous
import jax
import jax.numpy as jnp
from jax import lax
from jax.experimental import pallas as pl
from jax.experimental.pallas import tpu as pltpu
from jax.experimental.pallas import tpu_sc as plsc

MAX_LEN = 200
EMB_DIM = 32
BATCH = 4096
N_IDX = BATCH * MAX_LEN
N_GRP = MAX_LEN // 4
N_ROWS = N_IDX // 4
VOCAB = 1000000
TROWS = VOCAB // 4

_info = plsc.get_sparse_core_info()
NC, NS = _info.num_cores, _info.num_subcores
NW = NC * NS

TCHUNK = 768
N_TCHUNKS = 999936 // TCHUNK
REM = VOCAB - N_TCHUNKS * TCHUNK
TPAIRS = (N_TCHUNKS // NW + 2) // 2


def _transpose_compute(in_v, out_v, b):
    lane = lax.iota(jnp.int32, 16)

    @plsc.parallel_loop(0, TCHUNK // 4, unroll=2)
    def _(r):
        for q in range(4):
            j = 4 * r + q
            for h in range(2):
                v = plsc.load_gather(
                    in_v, [lane * 0 + b, 16 * h + lane, lane * 0 + j]
                )
                out_v[b, r, pl.ds(q * EMB_DIM + 16 * h, 16)] = v


def _transpose_body(tt_hbm, tail_hbm, out_hbm, in_v, out_v, sin, sout):
    wid = lax.axis_index("s") * NC + lax.axis_index("c")
    myn = (N_TCHUNKS - 1 - wid) // NW + 1

    def in_desc(j, b):
        c = j * NW + wid
        return pltpu.make_async_copy(
            tt_hbm.at[:, pl.ds(c * TCHUNK, TCHUNK)],
            in_v.at[b, :, pl.ds(0, TCHUNK)],
            sin,
        )

    def out_desc(j, b):
        c = j * NW + wid
        return pltpu.make_async_copy(
            out_v.at[b],
            out_hbm.at[pl.ds(c * (TCHUNK // 4), TCHUNK // 4)],
            sout,
        )

    in_desc(0, 0).start()

    def pair_body(j2, carry):
        for b in (0, 1):
            j = j2 * 2 + b

            @pl.when(j < myn)
            def _():
                in_desc(j, b).wait()

                @pl.when(j + 1 < myn)
                def _():
                    in_desc(j + 1, 1 - b).start()

                @pl.when(j >= 2)
                def _():
                    out_desc(j - 2, b).wait()

                _transpose_compute(in_v, out_v, b)
                out_desc(j, b).start()

        return carry

    lax.fori_loop(0, TPAIRS, pair_body, 0)
    out_desc(0, 0).wait()
    out_desc(0, 1).wait()

    @pl.when(wid == 17)
    def _():
        pltpu.sync_copy(tail_hbm, out_v.at[0, pl.ds(0, REM // 4)])
        pltpu.sync_copy(
            out_v.at[0, pl.ds(0, REM // 4)],
            out_hbm.at[pl.ds((VOCAB - REM) // 4, REM // 4)],
        )


def _sc_transpose(tableT, tail128):
    mesh = plsc.VectorSubcoreMesh(core_axis_name="c", subcore_axis_name="s")
    kern = pl.kernel(
        _transpose_body,
        mesh=mesh,
        out_type=jax.ShapeDtypeStruct((TROWS, 128), jnp.float32),
        scratch_types=[
            pltpu.VMEM((2, EMB_DIM, TCHUNK + 1), jnp.float32),
            pltpu.VMEM((2, TCHUNK // 4, 128), jnp.float32),
            pltpu.SemaphoreType.DMA,
            pltpu.SemaphoreType.DMA,
        ],
        compiler_params=pltpu.CompilerParams(
            use_tc_tiling_on_sc=True, needs_layout_passes=False
        ),
    )
    return kern(tableT, tail128)


BPW = BATCH // NW
TPC = 8
GITER = MAX_LEN // TPC
GN = TPC * BPW


def _gather_body(x_hbm, table_hbm, out_hbm, idx2_v, idxi_v, rows_v, sem):
    wid = lax.axis_index("s") * NC + lax.axis_index("c")
    lane = lax.iota(jnp.int32, 16)
    b0 = wid * BPW

    def chunk_body(i, carry):
        pltpu.sync_copy(
            x_hbm.at[pl.ds(i * TPC, TPC), pl.ds(b0, BPW)],
            idx2_v.at[:, pl.ds(0, BPW)],
        )

        @plsc.parallel_loop(0, GN // 16, unroll=2)
        def _(g2):
            n0 = g2 * 16
            kh = n0 >> 9
            rvec = kh * 4 + (lane & 3)
            cvec = ((n0 & 511) >> 2) + (lane >> 2)
            idxi_v[pl.ds(n0, 16)] = plsc.load_gather(idx2_v, [rvec, cvec])

        pltpu.async_copy(table_hbm.at[idxi_v], rows_v, sem).wait()
        for kh in range(2):
            k = 2 * i + kh
            pltpu.sync_copy(
                rows_v.at[pl.ds(kh * 512, 512)],
                out_hbm.at[pl.ds(k * (4 * BATCH) + 4 * b0, 512)],
            )
        return carry

    lax.fori_loop(0, GITER, chunk_body, 0)


def _sc_gather(x2d, table_lin):
    mesh = plsc.VectorSubcoreMesh(core_axis_name="c", subcore_axis_name="s")
    kern = pl.kernel(
        _gather_body,
        mesh=mesh,
        out_type=jax.ShapeDtypeStruct((N_IDX, EMB_DIM), jnp.float32),
        scratch_types=[
            pltpu.VMEM((TPC, BPW + 4), jnp.int32),
            pltpu.VMEM((GN,), jnp.int32),
            pltpu.VMEM((GN, EMB_DIM), jnp.float32),
            pltpu.SemaphoreType.DMA,
        ],
        compiler_params=pltpu.CompilerParams(
            use_tc_tiling_on_sc=False, needs_layout_passes=False
        ),
    )
    return kern(x2d, table_lin)


KPER = 5
NSTEP = N_GRP // KPER


def _mlp_body(emb_ref, w1_ref, b1_ref, w2_ref, b2_ref, out_ref, acc_ref):
    k0 = pl.program_id(0)

    @pl.when(k0 == 0)
    def _():
        acc_ref[...] = jnp.zeros_like(acc_ref)

    acc = acc_ref[...]
    for s in range(KPER):
        acc += jnp.dot(
            emb_ref[pl.ds(s * BATCH, BATCH), :],
            w1_ref[pl.ds(s * 128, 128), :],
            preferred_element_type=jnp.float32,
        )
    acc_ref[...] = acc

    @pl.when(k0 == NSTEP - 1)
    def _():
        h = jnp.maximum(acc + b1_ref[...], 0.0)
        o = jnp.dot(h, w2_ref[...], preferred_element_type=jnp.float32)
        out_ref[...] = jax.nn.sigmoid(o + b2_ref[...])


def _tc_mlp(emb128, w1t, b1, w2t, b2):
    f = pl.pallas_call(
        _mlp_body,
        grid=(NSTEP,),
        in_specs=[
            pl.BlockSpec((KPER * BATCH, 128), lambda k0: (k0, 0)),
            pl.BlockSpec((KPER * 128, 32), lambda k0: (k0, 0)),
            pl.BlockSpec((1, 32), lambda k0: (0, 0)),
            pl.BlockSpec((32, 1), lambda k0: (0, 0)),
            pl.BlockSpec((1, 1), lambda k0: (0, 0)),
        ],
        out_specs=pl.BlockSpec((BATCH, 1), lambda k0: (0, 0)),
        out_shape=jax.ShapeDtypeStruct((BATCH, 1), jnp.float32),
        scratch_shapes=[pltpu.VMEM((BATCH, 32), jnp.float32)],
    )
    return f(emb128, w1t, b1, w2t, b2)


@jax.jit
def kernel(x, table, W1, b1, W2, b2):
    tail128 = table[VOCAB - REM :].reshape(REM // 4, 128)
    tbl128 = _sc_transpose(table.T, tail128)
    table_lin = tbl128.reshape(VOCAB, EMB_DIM)
    emb128 = _sc_gather(x.T, table_lin).reshape(N_ROWS, 128)
    return _tc_mlp(emb128, W1.T, b1.reshape(1, 32), W2.T, b2.reshape(1, 1))
